# initial kernel scaffold (unmeasured)
import functools

import jax
import jax.numpy as jnp
from jax import lax
from jax.experimental import pallas as pl
from jax.experimental.pallas import tpu as pltpu

N_Z = 4
NEG_INF = -1e30


def kernel(Q, K, V, bt, lens):
    B, _, H, D = Q.shape
    P_loc, BS, _, _ = K.shape
    NB = bt.shape[1]
    T = P_loc * BS

    lens2 = lens.reshape(B, 1)

    def body(q_ref, k_ref, v_ref, bt_ref, lens_ref, out_ref,
             comm_o, comm_ml, send_o, recv_o, send_ml, recv_ml):
        my_x = lax.axis_index("x")
        my_y = lax.axis_index("y")
        my_z = lax.axis_index("z")

        bt_v = bt_ref[...]
        lens_v = lens_ref[...]
        jidx = lax.broadcasted_iota(jnp.int32, (B, NB), 1)
        valid = jidx < lens_v
        loc = bt_v - my_z * P_loc
        pidx = lax.broadcasted_iota(jnp.int32, (B, NB, P_loc), 2)
        hits = (loc[:, :, None] == pidx) & valid[:, :, None]
        counts = jnp.sum(hits.astype(jnp.float32), axis=1)
        w = jnp.broadcast_to(
            counts[:, :, None], (B, P_loc, BS)
        ).reshape(B, T)

        q = q_ref[:, 0, :, :]
        k = k_ref[...].reshape(T, H, D)
        v = v_ref[...].reshape(T, H, D)
        S = lax.dot_general(
            q, k, (((2,), (2,)), ((1,), (1,))),
            preferred_element_type=jnp.float32,
        )
        S = S * (D ** -0.5)
        Sm = jnp.where((w > 0)[None, :, :], S, NEG_INF)
        m = jnp.max(Sm, axis=2)
        e = jnp.exp(Sm - m[:, :, None]) * w[None, :, :]
        l = jnp.sum(e, axis=2)
        o = lax.dot_general(
            e, v, (((2,), (0,)), ((0,), (1,))),
            preferred_element_type=jnp.float32,
        )

        comm_o[0] = o
        comm_ml[0, 0] = m
        comm_ml[0, 1] = l

        rdmas = []
        for d in (1, 2, 3):
            peer = (my_z + d) % N_Z
            slot = N_Z - d
            tgt = (my_x, my_y, peer)
            r_o = pltpu.make_async_remote_copy(
                src_ref=comm_o.at[0],
                dst_ref=comm_o.at[slot],
                send_sem=send_o.at[d],
                recv_sem=recv_o.at[slot],
                device_id=tgt,
                device_id_type=pl.DeviceIdType.MESH,
            )
            r_ml = pltpu.make_async_remote_copy(
                src_ref=comm_ml.at[0],
                dst_ref=comm_ml.at[slot],
                send_sem=send_ml.at[d],
                recv_sem=recv_ml.at[slot],
                device_id=tgt,
                device_id_type=pl.DeviceIdType.MESH,
            )
            r_o.start()
            r_ml.start()
            rdmas += [r_o, r_ml]
        for r in rdmas:
            r.wait()

        m_all = comm_ml[:, 0]
        l_all = comm_ml[:, 1]
        o_all = comm_o[...]
        M = jnp.max(m_all, axis=0)
        coef = jnp.exp(m_all - M[None])
        l_tot = jnp.sum(l_all * coef, axis=0)
        o_tot = jnp.sum(o_all * coef[:, :, :, None], axis=0)
        res = o_tot / l_tot[:, :, None]
        out_ref[:, 0, :, :] = jnp.transpose(res, (1, 0, 2))

    return pl.pallas_call(
        body,
        out_shape=jax.ShapeDtypeStruct((B, 1, H, D), jnp.float32),
        in_specs=[pl.BlockSpec(memory_space=pltpu.VMEM)] * 5,
        out_specs=pl.BlockSpec(memory_space=pltpu.VMEM),
        scratch_shapes=[
            pltpu.VMEM((N_Z, H, B, D), jnp.float32),
            pltpu.VMEM((N_Z, 2, H, B), jnp.float32),
            pltpu.SemaphoreType.DMA((N_Z,)),
            pltpu.SemaphoreType.DMA((N_Z,)),
            pltpu.SemaphoreType.DMA((N_Z,)),
            pltpu.SemaphoreType.DMA((N_Z,)),
        ],
        compiler_params=pltpu.CompilerParams(collective_id=0),
    )(Q, K, V, bt, lens2)


# baseline (device time: 83109 ns/iter reference)
import jax
import jax.numpy as jnp
from jax import lax
from jax.experimental import pallas as pl
from jax.experimental.pallas import tpu as pltpu

N_Z = 4
NEG_INF = -1e30


def kernel(Q, K, V, bt, lens):
    B, _, H, D = Q.shape
    P_loc, BS, _, _ = K.shape
    NB = bt.shape[1]
    T = P_loc * BS

    lens2 = lens.reshape(B, 1)

    def body(q_ref, k_ref, v_ref, bt_ref, lens_ref, out_ref,
             comm_o, comm_ml, send_o, recv_o, send_ml, recv_ml):
        my_x = lax.axis_index("x")
        my_y = lax.axis_index("y")
        my_z = lax.axis_index("z")

        bt_v = bt_ref[...]
        lens_v = lens_ref[...]
        loc = bt_v - my_z * P_loc
        pidx = lax.broadcasted_iota(jnp.int32, (B, P_loc, NB), 1)
        jidx = lax.broadcasted_iota(jnp.int32, (B, P_loc, NB), 2)
        hits = (loc[:, None, :] == pidx) & (jidx < lens_v[:, None, :])
        counts = jnp.sum(hits.astype(jnp.float32), axis=2)

        ep = lax.broadcasted_iota(jnp.int32, (P_loc, T), 0)
        et = lax.broadcasted_iota(jnp.int32, (P_loc, T), 1) // BS
        expand = (ep == et).astype(jnp.float32)
        w = lax.dot_general(
            counts, expand, (((1,), (0,)), ((), ())),
            preferred_element_type=jnp.float32,
        )

        wmask = w > 0.0
        scale = D ** -0.5
        for h in range(H):
            q_h = q_ref[:, 0, h, :]
            k_h = k_ref[:, :, h, :].reshape(T, D)
            v_h = v_ref[:, :, h, :].reshape(T, D)
            S = lax.dot_general(
                q_h, k_h, (((1,), (1,)), ((), ())),
                preferred_element_type=jnp.float32,
            ) * scale
            Sm = jnp.where(wmask, S, NEG_INF)
            m = jnp.max(Sm, axis=1, keepdims=True)
            e = jnp.exp(Sm - m) * w
            l = jnp.sum(e, axis=1, keepdims=True)
            o = lax.dot_general(
                e, v_h, (((1,), (0,)), ((), ())),
                preferred_element_type=jnp.float32,
            )
            comm_o[0, h] = o
            comm_ml[0, 0, h] = m
            comm_ml[0, 1, h] = l

        rdmas = []
        for d in (1, 2, 3):
            peer = (my_z + d) % N_Z
            slot = N_Z - d
            tgt = (my_x, my_y, peer)
            r_o = pltpu.make_async_remote_copy(
                src_ref=comm_o.at[0],
                dst_ref=comm_o.at[slot],
                send_sem=send_o.at[d],
                recv_sem=recv_o.at[slot],
                device_id=tgt,
                device_id_type=pl.DeviceIdType.MESH,
            )
            r_ml = pltpu.make_async_remote_copy(
                src_ref=comm_ml.at[0],
                dst_ref=comm_ml.at[slot],
                send_sem=send_ml.at[d],
                recv_sem=recv_ml.at[slot],
                device_id=tgt,
                device_id_type=pl.DeviceIdType.MESH,
            )
            r_o.start()
            r_ml.start()
            rdmas += [r_o, r_ml]
        for r in rdmas:
            r.wait()

        m_all = comm_ml[:, 0]
        l_all = comm_ml[:, 1]
        o_all = comm_o[...]
        M = jnp.max(m_all, axis=0)
        coef = jnp.exp(m_all - M[None])
        l_tot = jnp.sum(l_all * coef, axis=0)
        o_tot = jnp.sum(o_all * coef, axis=0)
        res = o_tot / l_tot
        out_ref[:, 0, :, :] = jnp.transpose(res, (1, 0, 2))

    return pl.pallas_call(
        body,
        out_shape=jax.ShapeDtypeStruct((B, 1, H, D), jnp.float32),
        in_specs=[pl.BlockSpec(memory_space=pltpu.VMEM)] * 5,
        out_specs=pl.BlockSpec(memory_space=pltpu.VMEM),
        scratch_shapes=[
            pltpu.VMEM((N_Z, H, B, D), jnp.float32),
            pltpu.VMEM((N_Z, 2, H, B, 1), jnp.float32),
            pltpu.SemaphoreType.DMA((N_Z,)),
            pltpu.SemaphoreType.DMA((N_Z,)),
            pltpu.SemaphoreType.DMA((N_Z,)),
            pltpu.SemaphoreType.DMA((N_Z,)),
        ],
    )(Q, K, V, bt, lens2)
